# Initial kernel scaffold; baseline (speedup 1.0000x reference)
import jax
import jax.numpy as jnp
from jax.experimental import pallas as pl


def kernel(x, edge_index, weight, W_ih, W_hh, b_ih, b_hh, lin_W, lin_b):
    raise NotImplementedError("write your pallas kernel here")


# speedup vs baseline: 4.9061x; 4.9061x over previous
import functools

import jax
import jax.numpy as jnp
from jax import lax
from jax.experimental import pallas as pl
from jax.experimental.pallas import tpu as pltpu
from jax.experimental.pallas import tpu_sc as plsc

N = 10000
F = 128
E = 320000
NC = 2
NS = 16
NW = NC * NS
C = 128
CH = 79
EPAD = NW * CH * C
AGG_ROWS = 10240
DUMP = N
ROWS_PER_SUB = AGG_ROWS // NS


def _sc_scatter_fn(m_hbm, src_hbm, dst_hbm, out_hbm,
                   src_v, dst_v, rows_v, zero_v, acc_sh, gsem):
    cid = lax.axis_index("c")
    sid = lax.axis_index("s")
    wid = sid * NC + cid

    zvec = jnp.zeros((16,), jnp.float32)
    for r in range(16):
        for cc in range(F // 16):
            zero_v[r, pl.ds(cc * 16, 16)] = zvec

    base = sid * ROWS_PER_SUB

    def zbody(t, carry):
        pltpu.sync_copy(zero_v, acc_sh.at[pl.ds(base + t * 16, 16)])
        return carry
    lax.fori_loop(0, ROWS_PER_SUB // 16, zbody, 0)
    plsc.subcore_barrier()

    pltpu.sync_copy(src_hbm.at[wid], src_v)
    pltpu.sync_copy(dst_hbm.at[wid], dst_v)

    def body(j, carry):
        pltpu.async_copy(m_hbm.at[src_v.at[j]], rows_v.at[0], gsem).wait()
        pltpu.sync_copy(rows_v.at[0], acc_sh.at[dst_v.at[j]], add=True)
        return carry
    lax.fori_loop(0, CH, body, 0)
    plsc.subcore_barrier()

    def obody(t, carry):
        r0 = base + t * 128
        pltpu.sync_copy(acc_sh.at[pl.ds(r0, 128)],
                        out_hbm.at[cid, pl.ds(r0, 128)])
        return carry
    lax.fori_loop(0, ROWS_PER_SUB // 128, obody, 0)


_sc_scatter = pl.kernel(
    _sc_scatter_fn,
    out_type=jax.ShapeDtypeStruct((NC, AGG_ROWS, F), jnp.float32),
    mesh=plsc.VectorSubcoreMesh(core_axis_name="c", subcore_axis_name="s"),
    scratch_types=[
        pltpu.VMEM((CH, C), jnp.int32),
        pltpu.VMEM((CH, C), jnp.int32),
        pltpu.VMEM((1, C, F), jnp.float32),
        pltpu.VMEM((16, F), jnp.float32),
        pltpu.VMEM_SHARED((AGG_ROWS, F), jnp.float32),
        pltpu.SemaphoreType.DMA,
    ],
)


_DN = (((1,), (0,)), ((), ()))
R = 1000
GRID = N // R


def _mm_body(h_ref, w_ref, o_ref):
    o_ref[...] = lax.dot_general(h_ref[...], w_ref[...], _DN,
                                 preferred_element_type=jnp.float32)


def _first_mm(h, w):
    return pl.pallas_call(
        _mm_body,
        grid=(GRID,),
        in_specs=[pl.BlockSpec((R, F), lambda i: (i, 0)),
                  pl.BlockSpec((F, F), lambda i: (0, 0))],
        out_specs=pl.BlockSpec((R, F), lambda i: (i, 0)),
        out_shape=jax.ShapeDtypeStruct((N, F), jnp.float32),
    )(h, w)


def _gru_mid_body(p0_ref, p1_ref, h_ref, wih_ref, whh_ref, bih_ref, bhh_ref,
                  wn_ref, hnew_ref, mnext_ref):
    agg = p0_ref[0] + p1_ref[0]
    h = h_ref[...]
    gi = lax.dot_general(agg, wih_ref[...], _DN,
                         preferred_element_type=jnp.float32) + bih_ref[...]
    gh = lax.dot_general(h, whh_ref[...], _DN,
                         preferred_element_type=jnp.float32) + bhh_ref[...]
    r = jax.nn.sigmoid(gi[:, :F] + gh[:, :F])
    z = jax.nn.sigmoid(gi[:, F:2 * F] + gh[:, F:2 * F])
    n = jnp.tanh(gi[:, 2 * F:] + r * gh[:, 2 * F:])
    hn = (1.0 - z) * n + z * h
    hnew_ref[...] = hn
    mnext_ref[...] = lax.dot_general(hn, wn_ref[...], _DN,
                                     preferred_element_type=jnp.float32)


def _gru_mid(parts, h, wihT, whhT, bih2, bhh2, w_next):
    return pl.pallas_call(
        _gru_mid_body,
        grid=(GRID,),
        in_specs=[pl.BlockSpec((1, R, F), lambda i: (0, i, 0)),
                  pl.BlockSpec((1, R, F), lambda i: (1, i, 0)),
                  pl.BlockSpec((R, F), lambda i: (i, 0)),
                  pl.BlockSpec((F, 3 * F), lambda i: (0, 0)),
                  pl.BlockSpec((F, 3 * F), lambda i: (0, 0)),
                  pl.BlockSpec((1, 3 * F), lambda i: (0, 0)),
                  pl.BlockSpec((1, 3 * F), lambda i: (0, 0)),
                  pl.BlockSpec((F, F), lambda i: (0, 0))],
        out_specs=[pl.BlockSpec((R, F), lambda i: (i, 0)),
                   pl.BlockSpec((R, F), lambda i: (i, 0))],
        out_shape=[jax.ShapeDtypeStruct((N, F), jnp.float32),
                   jax.ShapeDtypeStruct((N, F), jnp.float32)],
    )(parts, parts, h, wihT, whhT, bih2, bhh2, w_next)


def _gru_last_body(p0_ref, p1_ref, h_ref, wih_ref, whh_ref, bih_ref, bhh_ref,
                   lw_ref, lb_ref, out_ref):
    agg = p0_ref[0] + p1_ref[0]
    h = h_ref[...]
    gi = lax.dot_general(agg, wih_ref[...], _DN,
                         preferred_element_type=jnp.float32) + bih_ref[...]
    gh = lax.dot_general(h, whh_ref[...], _DN,
                         preferred_element_type=jnp.float32) + bhh_ref[...]
    r = jax.nn.sigmoid(gi[:, :F] + gh[:, :F])
    z = jax.nn.sigmoid(gi[:, F:2 * F] + gh[:, F:2 * F])
    n = jnp.tanh(gi[:, 2 * F:] + r * gh[:, 2 * F:])
    hn = (1.0 - z) * n + z * h
    hr = jnp.maximum(hn, 0.0)
    out_ref[...] = lax.dot_general(hr, lw_ref[...], _DN,
                                   preferred_element_type=jnp.float32) + lb_ref[...]


def _gru_last(parts, h, wihT, whhT, bih2, bhh2, lwT, lb2):
    return pl.pallas_call(
        _gru_last_body,
        grid=(GRID,),
        in_specs=[pl.BlockSpec((1, R, F), lambda i: (0, i, 0)),
                  pl.BlockSpec((1, R, F), lambda i: (1, i, 0)),
                  pl.BlockSpec((R, F), lambda i: (i, 0)),
                  pl.BlockSpec((F, 3 * F), lambda i: (0, 0)),
                  pl.BlockSpec((F, 3 * F), lambda i: (0, 0)),
                  pl.BlockSpec((1, 3 * F), lambda i: (0, 0)),
                  pl.BlockSpec((1, 3 * F), lambda i: (0, 0)),
                  pl.BlockSpec((F, 1), lambda i: (0, 0)),
                  pl.BlockSpec((1, 1), lambda i: (0, 0))],
        out_specs=pl.BlockSpec((R, 1), lambda i: (i, 0)),
        out_shape=jax.ShapeDtypeStruct((N, 1), jnp.float32),
    )(parts, parts, h, wihT, whhT, bih2, bhh2, lwT, lb2)


def kernel(x, edge_index, weight, W_ih, W_hh, b_ih, b_hh, lin_W, lin_b):
    src = edge_index[0].astype(jnp.int32)
    dst = edge_index[1].astype(jnp.int32)
    src_w = jnp.concatenate(
        [src, jnp.zeros((EPAD - E,), jnp.int32)]).reshape(NW, CH, C)
    dst_w = jnp.concatenate(
        [dst, jnp.full((EPAD - E,), DUMP, jnp.int32)]).reshape(NW, CH, C)

    wihT = W_ih.T
    whhT = W_hh.T
    bih2 = b_ih.reshape(1, 3 * F)
    bhh2 = b_hh.reshape(1, 3 * F)
    lwT = lin_W.T
    lb2 = lin_b.reshape(1, 1)

    h = x
    m = _first_mm(h, weight[0])
    for i in range(2):
        parts = _sc_scatter(m, src_w, dst_w)
        h, m = _gru_mid(parts, h, wihT, whhT, bih2, bhh2, weight[i + 1])
    parts = _sc_scatter(m, src_w, dst_w)
    return _gru_last(parts, h, wihT, whhT, bih2, bhh2, lwT, lb2)
